# Initial kernel scaffold; baseline (speedup 1.0000x reference)
import jax
import jax.numpy as jnp
from jax.experimental import pallas as pl


def kernel(input_predicted, input_encoded, mask_ids):
    raise NotImplementedError("write your pallas kernel here")


# speedup vs baseline: 169.4006x; 169.4006x over previous
import functools

import numpy as np
import jax
import jax.numpy as jnp
from jax import lax
from jax.experimental import pallas as pl
from jax.experimental.pallas import tpu as pltpu
from jax.experimental.pallas import tpu_sc as plsc

_N_FALSE_NEGATIVES = 100
_TEMPERATURE = 0.1

_ROT = (13, 15, 26, 6, 17, 29, 16, 24)


def _np_threefry2x32(ks0, ks1, x0, x1):
    def rotl(x, n):
        return ((x << np.uint32(n)) | (x >> np.uint32(32 - n))).astype(np.uint32)

    x0 = np.asarray(x0, np.uint32).copy()
    x1 = np.asarray(x1, np.uint32).copy()
    ks2 = np.uint32(np.uint32(ks0) ^ np.uint32(ks1) ^ np.uint32(0x1BD11BDA))
    ks = (np.uint32(ks0), np.uint32(ks1), ks2)
    x0 = (x0 + ks[0]).astype(np.uint32)
    x1 = (x1 + ks[1]).astype(np.uint32)
    for i in range(5):
        for r in (_ROT[0:4] if i % 2 == 0 else _ROT[4:8]):
            x0 = (x0 + x1).astype(np.uint32)
            x1 = rotl(x1, r)
            x1 = (x1 ^ x0).astype(np.uint32)
        x0 = (x0 + ks[(i + 1) % 3]).astype(np.uint32)
        x1 = (x1 + ks[(i + 2) % 3] + np.uint32(i + 1)).astype(np.uint32)
    return x0, x1


def _np_random_bits32(ks0, ks1, size):
    idx = np.arange(size, dtype=np.uint64)
    c1 = (idx >> np.uint64(32)).astype(np.uint32)
    c2 = (idx & np.uint64(0xFFFFFFFF)).astype(np.uint32)
    b1, b2 = _np_threefry2x32(ks0, ks1, c1, c2)
    return b1 ^ b2


def _np_randint(seed, shape, lo, hi):
    ks0 = np.uint32((seed >> 32) & 0xFFFFFFFF)
    ks1 = np.uint32(seed & 0xFFFFFFFF)
    b1, b2 = _np_threefry2x32(ks0, ks1, np.zeros(2, np.uint32),
                              np.arange(2, dtype=np.uint32))
    size = int(np.prod(shape))
    y = _np_random_bits32(b1[0], b2[0], size)
    z = _np_random_bits32(b1[1], b2[1], size)
    span = np.uint32(hi - lo)
    m16 = np.uint32(np.uint32(65536) % span)
    mult = np.uint32((np.uint64(m16) * np.uint64(m16)) % span)
    q = (((y % span) * mult).astype(np.uint32) + (z % span)).astype(np.uint32) % span
    return (np.int64(lo) + q).astype(np.int32).reshape(shape)


@functools.lru_cache(maxsize=None)
def _neg_count_matrix(m: int) -> np.ndarray:
    d = _np_randint(42, (m, _N_FALSE_NEGATIVES), 0, m - 2)
    i = np.arange(m, dtype=np.int64)[:, None]
    seq2 = d.astype(np.int64) + (d >= i)
    flat = (np.arange(m, dtype=np.int64)[:, None] * m + seq2).ravel()
    counts = np.bincount(flat, minlength=m * m).reshape(m, m)
    return counts.astype(np.int8)


def _sc_gather_rows(table_a, table_b, mask_flat, t_stride):
    m2 = mask_flat.shape[0]
    m = m2 // 2
    d = table_a.shape[1]
    info = plsc.get_sparse_core_info()
    nc, ns, nl = info.num_cores, info.num_subcores, info.num_lanes
    nw = nc * ns
    chunk = m // nw
    assert m % nw == 0 and chunk % nl == 0 and d % nl == 0

    mesh = plsc.VectorSubcoreMesh(core_axis_name="c", subcore_axis_name="s")
    out_type = (
        jax.ShapeDtypeStruct((m, d), jnp.float32),
        jax.ShapeDtypeStruct((m, d), jnp.float32),
    )

    @functools.partial(
        pl.kernel,
        mesh=mesh,
        out_type=out_type,
        compiler_params=pltpu.CompilerParams(needs_layout_passes=False),
        scratch_types=[
            pltpu.VMEM((2 * chunk,), jnp.int32),
            pltpu.VMEM((chunk,), jnp.int32),
            pltpu.VMEM((chunk, d), jnp.float32),
            pltpu.VMEM((chunk, d), jnp.float32),
            pltpu.SemaphoreType.DMA,
            pltpu.SemaphoreType.DMA,
        ],
    )
    def gather_kernel(a_hbm, b_hbm, mask_hbm, a_out, b_out,
                      mvec, idx_v, a_rows, b_rows, sem_a, sem_b):
        wid = lax.axis_index("s") * nc + lax.axis_index("c")
        base = wid * chunk
        pltpu.sync_copy(mask_hbm.at[pl.ds(base * 2, 2 * chunk)], mvec)
        lanes = lax.iota(jnp.int32, nl)
        for i in range(chunk // nl):
            off = 2 * nl * i
            rows = plsc.load_gather(mvec, [lanes * 2 + off])
            cols = plsc.load_gather(mvec, [lanes * 2 + off + 1])
            idx_v[pl.ds(nl * i, nl)] = rows * t_stride + cols
        cp_a = pltpu.async_copy(a_hbm.at[idx_v], a_rows, sem_a)
        cp_b = pltpu.async_copy(b_hbm.at[idx_v], b_rows, sem_b)
        cp_a.wait()
        cp_b.wait()
        pltpu.sync_copy(a_rows, a_out.at[pl.ds(base, chunk)])
        pltpu.sync_copy(b_rows, b_out.at[pl.ds(base, chunk)])

    return gather_kernel(table_a, table_b, mask_flat)


def _tc_loss(p_rows, e_rows, counts):
    m, d = e_rows.shape
    bm = 256
    grid = m // bm
    inv_m = np.float32(1.0 / m)

    def body(p_ref, e_ref, c_ref, loss_ref, acc_ref):
        r = pl.program_id(0)

        @pl.when(r == 0)
        def _init():
            loss_ref[0, 0] = jnp.float32(0.0)
            acc_ref[0, 0] = jnp.float32(0.0)

        e = e_ref[...]
        en = e / jnp.maximum(
            jnp.sqrt(jnp.sum(e * e, axis=1, keepdims=True)), 1e-12)
        p = p_ref[...]
        pn = p / jnp.maximum(
            jnp.sqrt(jnp.sum(p * p, axis=1, keepdims=True)), 1e-12)
        s = lax.dot_general(
            pn, en, (((1,), (1,)), ((), ())),
            preferred_element_type=jnp.float32,
        ) / _TEMPERATURE
        rows = r * bm + lax.broadcasted_iota(jnp.int32, (bm, m), 0)
        cols = lax.broadcasted_iota(jnp.int32, (bm, m), 1)
        pos = jnp.sum(jnp.where(rows == cols, s, 0.0), axis=1, keepdims=True)
        cf = c_ref[...].astype(jnp.float32)
        neg_max = jnp.max(
            jnp.where(cf > 0.0, s, -jnp.inf), axis=1, keepdims=True)
        mx = jnp.maximum(pos, neg_max)
        sumexp = jnp.exp(pos - mx) + jnp.sum(
            jnp.exp(s - mx) * cf, axis=1, keepdims=True)
        loss_vec = mx + jnp.log(sumexp) - pos
        acc_vec = (pos >= neg_max).astype(jnp.float32)
        loss_ref[0, 0] += jnp.sum(loss_vec) * inv_m
        acc_ref[0, 0] += jnp.sum(acc_vec) * inv_m

    loss, acc = pl.pallas_call(
        body,
        grid=(grid,),
        in_specs=[
            pl.BlockSpec((bm, d), lambda r: (r, 0)),
            pl.BlockSpec((m, d), lambda r: (0, 0)),
            pl.BlockSpec((bm, m), lambda r: (r, 0)),
        ],
        out_specs=[
            pl.BlockSpec(memory_space=pltpu.SMEM),
            pl.BlockSpec(memory_space=pltpu.SMEM),
        ],
        out_shape=[
            jax.ShapeDtypeStruct((1, 1), jnp.float32),
            jax.ShapeDtypeStruct((1, 1), jnp.float32),
        ],
        compiler_params=pltpu.CompilerParams(
            dimension_semantics=("arbitrary",)),
    )(p_rows, e_rows, counts)
    return loss[0, 0], acc[0, 0]


def kernel(input_predicted, input_encoded, mask_ids):
    b, t, d = input_encoded.shape
    m = mask_ids.shape[0]
    enc = input_encoded.reshape(b * t, d)
    pred = input_predicted.reshape(b * t, d)
    e_rows, p_rows = _sc_gather_rows(enc, pred, mask_ids.reshape(-1), t)
    counts = jnp.asarray(_neg_count_matrix(m))
    loss, acc = _tc_loss(p_rows, e_rows, counts)
    return loss, acc


# speedup vs baseline: 197.1815x; 1.1640x over previous
import functools

import numpy as np
import jax
import jax.numpy as jnp
from jax import lax
from jax.experimental import pallas as pl
from jax.experimental.pallas import tpu as pltpu
from jax.experimental.pallas import tpu_sc as plsc

_N_FALSE_NEGATIVES = 100
_TEMPERATURE = 0.1

_ROT = (13, 15, 26, 6, 17, 29, 16, 24)


def _np_threefry2x32(ks0, ks1, x0, x1):
    def rotl(x, n):
        return ((x << np.uint32(n)) | (x >> np.uint32(32 - n))).astype(np.uint32)

    x0 = np.asarray(x0, np.uint32).copy()
    x1 = np.asarray(x1, np.uint32).copy()
    ks2 = np.uint32(np.uint32(ks0) ^ np.uint32(ks1) ^ np.uint32(0x1BD11BDA))
    ks = (np.uint32(ks0), np.uint32(ks1), ks2)
    x0 = (x0 + ks[0]).astype(np.uint32)
    x1 = (x1 + ks[1]).astype(np.uint32)
    for i in range(5):
        for r in (_ROT[0:4] if i % 2 == 0 else _ROT[4:8]):
            x0 = (x0 + x1).astype(np.uint32)
            x1 = rotl(x1, r)
            x1 = (x1 ^ x0).astype(np.uint32)
        x0 = (x0 + ks[(i + 1) % 3]).astype(np.uint32)
        x1 = (x1 + ks[(i + 2) % 3] + np.uint32(i + 1)).astype(np.uint32)
    return x0, x1


def _np_random_bits32(ks0, ks1, size):
    idx = np.arange(size, dtype=np.uint64)
    c1 = (idx >> np.uint64(32)).astype(np.uint32)
    c2 = (idx & np.uint64(0xFFFFFFFF)).astype(np.uint32)
    b1, b2 = _np_threefry2x32(ks0, ks1, c1, c2)
    return b1 ^ b2


def _np_randint(seed, shape, lo, hi):
    ks0 = np.uint32((seed >> 32) & 0xFFFFFFFF)
    ks1 = np.uint32(seed & 0xFFFFFFFF)
    b1, b2 = _np_threefry2x32(ks0, ks1, np.zeros(2, np.uint32),
                              np.arange(2, dtype=np.uint32))
    size = int(np.prod(shape))
    y = _np_random_bits32(b1[0], b2[0], size)
    z = _np_random_bits32(b1[1], b2[1], size)
    span = np.uint32(hi - lo)
    m16 = np.uint32(np.uint32(65536) % span)
    mult = np.uint32((np.uint64(m16) * np.uint64(m16)) % span)
    q = (((y % span) * mult).astype(np.uint32) + (z % span)).astype(np.uint32) % span
    return (np.int64(lo) + q).astype(np.int32).reshape(shape)


@functools.lru_cache(maxsize=None)
def _neg_count_matrix(m: int) -> np.ndarray:
    d = _np_randint(42, (m, _N_FALSE_NEGATIVES), 0, m - 2)
    i = np.arange(m, dtype=np.int64)[:, None]
    seq2 = d.astype(np.int64) + (d >= i)
    flat = (np.arange(m, dtype=np.int64)[:, None] * m + seq2).ravel()
    counts = np.bincount(flat, minlength=m * m).reshape(m, m)
    return counts.astype(np.int8)


def _sc_gather_rows(table_a, table_b, mask_flat, t_stride):
    m2 = mask_flat.shape[0]
    m = m2 // 2
    d = table_a.shape[1]
    info = plsc.get_sparse_core_info()
    nc, ns, nl = info.num_cores, info.num_subcores, info.num_lanes
    nw = nc * ns
    chunk = m // nw
    assert m % nw == 0 and chunk % nl == 0 and d % nl == 0

    mesh = plsc.VectorSubcoreMesh(core_axis_name="c", subcore_axis_name="s")
    out_type = (
        jax.ShapeDtypeStruct((m, d), jnp.float32),
        jax.ShapeDtypeStruct((m, d), jnp.float32),
    )

    @functools.partial(
        pl.kernel,
        mesh=mesh,
        out_type=out_type,
        compiler_params=pltpu.CompilerParams(needs_layout_passes=False),
        scratch_types=[
            pltpu.VMEM((2 * chunk,), jnp.int32),
            pltpu.VMEM((chunk,), jnp.int32),
            pltpu.VMEM((chunk, d), jnp.float32),
            pltpu.VMEM((chunk, d), jnp.float32),
            pltpu.SemaphoreType.DMA,
            pltpu.SemaphoreType.DMA,
        ],
    )
    def gather_kernel(a_hbm, b_hbm, mask_hbm, a_out, b_out,
                      mvec, idx_v, a_rows, b_rows, sem_a, sem_b):
        wid = lax.axis_index("s") * nc + lax.axis_index("c")
        base = wid * chunk
        pltpu.sync_copy(mask_hbm.at[pl.ds(base * 2, 2 * chunk)], mvec)
        lanes = lax.iota(jnp.int32, nl)
        for i in range(chunk // nl):
            off = 2 * nl * i
            rows = plsc.load_gather(mvec, [lanes * 2 + off])
            cols = plsc.load_gather(mvec, [lanes * 2 + off + 1])
            idx_v[pl.ds(nl * i, nl)] = rows * t_stride + cols
        cp_a = pltpu.async_copy(a_hbm.at[idx_v], a_rows, sem_a)
        cp_b = pltpu.async_copy(b_hbm.at[idx_v], b_rows, sem_b)
        cp_a.wait()
        cp_b.wait()
        pltpu.sync_copy(a_rows, a_out.at[pl.ds(base, chunk)])
        pltpu.sync_copy(b_rows, b_out.at[pl.ds(base, chunk)])

    return gather_kernel(table_a, table_b, mask_flat)


def _tc_loss(p_rows, e_rows, counts):
    m, d = e_rows.shape
    bm = 256
    grid = m // bm
    inv_m = np.float32(1.0 / m)

    def body(p_ref, e_ref, c_ref, loss_ref, acc_ref, en_ref):
        r = pl.program_id(0)

        @pl.when(r == 0)
        def _init():
            e = e_ref[...]
            en_ref[...] = e / jnp.maximum(
                jnp.sqrt(jnp.sum(e * e, axis=1, keepdims=True)), 1e-12)
            loss_ref[0, 0] = jnp.float32(0.0)
            acc_ref[0, 0] = jnp.float32(0.0)

        p = p_ref[...]
        pn = p / (jnp.maximum(
            jnp.sqrt(jnp.sum(p * p, axis=1, keepdims=True)), 1e-12)
            * _TEMPERATURE)
        s = lax.dot_general(
            pn, en_ref[...], (((1,), (1,)), ((), ())),
            preferred_element_type=jnp.float32,
        )
        en_diag = en_ref[pl.ds(r * bm, bm), :]
        pos = jnp.sum(pn * en_diag, axis=1, keepdims=True)
        cf = c_ref[...].astype(jnp.float32)
        neg_max = jnp.max(
            jnp.where(cf > 0.0, s, -jnp.inf), axis=1, keepdims=True)
        mx = jnp.maximum(pos, neg_max)
        sumexp = jnp.exp(pos - mx) + jnp.sum(
            jnp.exp(s - mx) * cf, axis=1, keepdims=True)
        loss_vec = mx + jnp.log(sumexp) - pos
        acc_vec = (pos >= neg_max).astype(jnp.float32)
        loss_ref[0, 0] += jnp.sum(loss_vec) * inv_m
        acc_ref[0, 0] += jnp.sum(acc_vec) * inv_m

    loss, acc = pl.pallas_call(
        body,
        grid=(grid,),
        in_specs=[
            pl.BlockSpec((bm, d), lambda r: (r, 0)),
            pl.BlockSpec((m, d), lambda r: (0, 0)),
            pl.BlockSpec((bm, m), lambda r: (r, 0)),
        ],
        out_specs=[
            pl.BlockSpec(memory_space=pltpu.SMEM),
            pl.BlockSpec(memory_space=pltpu.SMEM),
        ],
        out_shape=[
            jax.ShapeDtypeStruct((1, 1), jnp.float32),
            jax.ShapeDtypeStruct((1, 1), jnp.float32),
        ],
        scratch_shapes=[pltpu.VMEM((m, d), jnp.float32)],
        compiler_params=pltpu.CompilerParams(
            dimension_semantics=("arbitrary",)),
    )(p_rows, e_rows, counts)
    return loss[0, 0], acc[0, 0]


def kernel(input_predicted, input_encoded, mask_ids):
    b, t, d = input_encoded.shape
    m = mask_ids.shape[0]
    enc = input_encoded.reshape(b * t, d)
    pred = input_predicted.reshape(b * t, d)
    e_rows, p_rows = _sc_gather_rows(enc, pred, mask_ids.reshape(-1), t)
    counts = jnp.asarray(_neg_count_matrix(m))
    loss, acc = _tc_loss(p_rows, e_rows, counts)
    return loss, acc
